# Initial kernel scaffold; baseline (speedup 1.0000x reference)
import jax
import jax.numpy as jnp
from jax.experimental import pallas as pl


def kernel(scores, bbox_deltas, im_info):
    raise NotImplementedError("write your pallas kernel here")


# speedup vs baseline: 6.8474x; 6.8474x over previous
import numpy as np
import jax
import jax.numpy as jnp
from jax import lax
from jax.experimental import pallas as pl
from jax.experimental.pallas import tpu as pltpu

FEATURE_STRIDE = 16
PRE_NMS_TOPN = 6000
POST_NMS_TOPN = 300
NMS_THRESH = 0.7
MIN_SIZE = 16.0
NEG = -1e9

H = W = 32
A = 9
N = H * W * A
NB = N // 128
NS = 6144
NSB = NS // 128
NOUT = 384
_PREC = lax.Precision.HIGHEST


def _gen_anchors():
    scales = np.array([8.0, 16.0, 32.0])
    ratios = np.array([0.5, 1.0, 2.0])
    base = np.array([1.0, 1.0, 16.0, 16.0]) - 1.0
    w = base[2] - base[0] + 1.0
    h = base[3] - base[1] + 1.0
    xc = base[0] + 0.5 * (w - 1.0)
    yc = base[1] + 0.5 * (h - 1.0)
    size = w * h
    ws = np.round(np.sqrt(size / ratios))
    hs = np.round(ws * ratios)

    def mk(ws, hs):
        ws = ws[:, None]
        hs = hs[:, None]
        return np.hstack((xc - 0.5 * (ws - 1.0), yc - 0.5 * (hs - 1.0),
                          xc + 0.5 * (ws - 1.0), yc + 0.5 * (hs - 1.0)))

    ra = mk(ws, hs)
    outs = []
    for i in range(ra.shape[0]):
        aw = ra[i, 2] - ra[i, 0] + 1.0
        ah = ra[i, 3] - ra[i, 1] + 1.0
        axc = ra[i, 0] + 0.5 * (aw - 1.0)
        ayc = ra[i, 1] + 0.5 * (ah - 1.0)
        ws2 = (aw * scales)[:, None]
        hs2 = (ah * scales)[:, None]
        outs.append(np.hstack((axc - 0.5 * (ws2 - 1.0), ayc - 0.5 * (hs2 - 1.0),
                               axc + 0.5 * (ws2 - 1.0), ayc + 0.5 * (hs2 - 1.0))))
    anchors = np.vstack(outs).astype(np.float32)
    sx, sy = np.meshgrid(np.arange(W) * FEATURE_STRIDE, np.arange(H) * FEATURE_STRIDE)
    shifts = np.stack([sx.ravel(), sy.ravel(), sx.ravel(), sy.ravel()], 1).astype(np.float32)
    return (anchors[None, :, :] + shifts[:, None, :]).reshape(-1, 4)


_ALL_ANCHORS = _gen_anchors()


def _eye(n):
    r = lax.broadcasted_iota(jnp.int32, (n, n), 0)
    c = lax.broadcasted_iota(jnp.int32, (n, n), 1)
    return jnp.where(r == c, 1.0, 0.0).astype(jnp.float32)


def _t(x):
    e = _eye(x.shape[1])
    return lax.dot_general(e, x, (((1,), (1,)), ((), ())),
                           precision=_PREC, preferred_element_type=jnp.float32)


def _k1_body(ax1, ay1, ax2, ay2, dx, dy, dw, dh, raw, lims,
             rows_ref, rank_ref, sc_ref, x1_ref, y1_ref, x2_ref, y2_ref):
    widths = ax2[...] - ax1[...] + 1.0
    heights = ay2[...] - ay1[...] + 1.0
    ctrx = ax1[...] + 0.5 * widths
    ctry = ay1[...] + 0.5 * heights
    pcx = dx[...] * widths + ctrx
    pcy = dy[...] * heights + ctry
    pw = jnp.exp(dw[...]) * widths
    ph = jnp.exp(dh[...]) * heights
    x1 = pcx - 0.5 * pw
    y1 = pcy - 0.5 * ph
    x2 = pcx + 0.5 * pw
    y2 = pcy + 0.5 * ph
    lv = lims[...]
    wmax = lv[0:1, 0:1]
    hmax = lv[0:1, 1:2]
    msz = lv[0:1, 2:3]
    x1 = jnp.minimum(jnp.maximum(x1, 0.0), wmax)
    y1 = jnp.minimum(jnp.maximum(y1, 0.0), hmax)
    x2 = jnp.minimum(jnp.maximum(x2, 0.0), wmax)
    y2 = jnp.minimum(jnp.maximum(y2, 0.0), hmax)
    ws = x2 - x1 + 1.0
    hs = y2 - y1 + 1.0
    valid = (ws >= msz) & (hs >= msz)
    sc = jnp.where(valid, raw[...], NEG)
    sc_ref[...] = sc
    x1_ref[...] = x1
    y1_ref[...] = y1
    x2_ref[...] = x2
    y2_ref[...] = y2

    lane = lax.broadcasted_iota(jnp.int32, (1, 128), 1)
    sub = lax.broadcasted_iota(jnp.int32, (128, 1), 0)

    def rows_body(ib, _):
        c5 = jnp.concatenate([
            x1_ref[pl.ds(ib, 1), :],
            y1_ref[pl.ds(ib, 1), :],
            x2_ref[pl.ds(ib, 1), :],
            y2_ref[pl.ds(ib, 1), :],
            sc_ref[pl.ds(ib, 1), :],
        ], axis=0)
        t = _t(c5)
        tp = jnp.concatenate([t, jnp.zeros((128, 11), jnp.float32)], axis=1)
        rows_ref[pl.ds(ib * 128, 128), :] = tp
        return 0

    lax.fori_loop(0, NB, rows_body, 0)

    ltm = lane < sub

    def rank_body(ib, _):
        srow_i = sc_ref[pl.ds(ib, 1), :]
        scol = _t(srow_i)

        def ge_body(jb, acc):
            srow = sc_ref[pl.ds(jb, 1), :]
            return acc + jnp.sum(jnp.where(srow >= scol, 1.0, 0.0),
                                 axis=1, keepdims=True)

        def gt_body(jb, acc):
            srow = sc_ref[pl.ds(jb, 1), :]
            return acc + jnp.sum(jnp.where(srow > scol, 1.0, 0.0),
                                 axis=1, keepdims=True)

        acc = lax.fori_loop(0, ib, ge_body, jnp.zeros((128, 1), jnp.float32))
        acc = lax.fori_loop(ib + 1, NB, gt_body, acc)
        diag = (srow_i > scol) | ((srow_i == scol) & ltm)
        acc = acc + jnp.sum(jnp.where(diag, 1.0, 0.0), axis=1, keepdims=True)
        rrow = lax.dot_general(acc, _eye(128), (((0,), (0,)), ((), ())),
                               precision=_PREC,
                               preferred_element_type=jnp.float32)
        rank_ref[pl.ds(ib, 1), :] = rrow.astype(jnp.int32)
        return 0

    lax.fori_loop(0, NB, rank_body, 0)


def _k2_body(rows_ref, rank_ref, out_ref):
    sub = lax.broadcasted_iota(jnp.int32, (128, 1), 0)

    def rb_body(rb, _):
        col = rb * 128 + sub

        def jb_body(jb, acc):
            rrow = rank_ref[pl.ds(jb, 1), :]
            oh = jnp.where(rrow == col, 1.0, 0.0)
            vblk = rows_ref[pl.ds(jb * 128, 128), :]
            return acc + lax.dot_general(
                oh, vblk, (((1,), (0,)), ((), ())),
                precision=_PREC, preferred_element_type=jnp.float32)

        acc = lax.fori_loop(0, NB, jb_body, jnp.zeros((128, 16), jnp.float32))
        out_ref[pl.ds(rb * 128, 128), :] = acc
        return 0

    lax.fori_loop(0, NSB, rb_body, 0)


def _k3_body(in_ref, out_ref, rowsT_ref, w_ref):
    e16 = _eye(16)
    for b in range(NSB):
        blk = in_ref[b * 128:(b + 1) * 128, :]
        rowsT_ref[:, b * 128:(b + 1) * 128] = lax.dot_general(
            e16, blk, (((1,), (1,)), ((), ())),
            precision=_PREC, preferred_element_type=jnp.float32)

    x1r = rowsT_ref[0:1, :]
    y1r = rowsT_ref[1:2, :]
    x2r = rowsT_ref[2:3, :]
    y2r = rowsT_ref[3:4, :]
    areas_row = (x2r - x1r + 1.0) * (y2r - y1r + 1.0)

    lane1 = lax.broadcasted_iota(jnp.int32, (1, 128), 1)
    slot_full = lax.broadcasted_iota(jnp.int32, (1, NS), 1)
    supp = jnp.zeros((1, NS), jnp.float32)
    keepall = []

    for b in range(NSB):
        base = b * 128
        blk = in_ref[base:base + 128, :]
        bx1 = blk[:, 0:1]
        by1 = blk[:, 1:2]
        bx2 = blk[:, 2:3]
        by2 = blk[:, 3:4]
        areac = (bx2 - bx1 + 1.0) * (by2 - by1 + 1.0)
        xx1 = jnp.maximum(bx1, x1r[:, base:])
        yy1 = jnp.maximum(by1, y1r[:, base:])
        xx2 = jnp.minimum(bx2, x2r[:, base:])
        yy2 = jnp.minimum(by2, y2r[:, base:])
        iw = jnp.maximum(xx2 - xx1 + 1.0, 0.0)
        ih = jnp.maximum(yy2 - yy1 + 1.0, 0.0)
        inter = iw * ih
        iou = inter / (areac + areas_row[:, base:] - inter)
        w_ref[...] = iou[:, 0:128]

        validb = jnp.where(base + lane1 < PRE_NMS_TOPN, 1.0, 0.0)
        keep0 = validb * (1.0 - supp[:, base:base + 128])

        def greedy(i, keep):
            wrow = w_ref[pl.ds(i, 1), :]
            ki = jnp.max(jnp.where(lane1 == i, keep, 0.0))
            kill = (wrow > NMS_THRESH) & (lane1 > i) & (ki > 0.5)
            return jnp.where(kill, 0.0, keep)

        keep = lax.fori_loop(0, 128, greedy, keep0)
        keepall.append(keep)

        kc = _t(keep)
        supmat = jnp.where((iou > NMS_THRESH) & (kc > 0.5), 1.0, 0.0)
        supnew = jnp.max(supmat, axis=0, keepdims=True)
        if b < NSB - 1:
            tail = jnp.maximum(supp[:, base + 128:], supnew[:, 128:])
            supp = jnp.concatenate([supp[:, :base + 128], tail], axis=1)

    keepall = jnp.concatenate(keepall, axis=1)

    sc_row = rowsT_ref[4:5, :]
    slot_ok = jnp.where(slot_full < PRE_NMS_TOPN, 1.0, 0.0)
    goodf = keepall * jnp.where(sc_row != NEG, 1.0, 0.0) * slot_ok
    badf = (1.0 - goodf) * slot_ok

    lt = jnp.where(lax.broadcasted_iota(jnp.int32, (128, 128), 0) <
                   lax.broadcasted_iota(jnp.int32, (128, 128), 1), 1.0, 0.0)
    posg, posb = [], []
    og = jnp.zeros((1, 1), jnp.float32)
    ob = jnp.zeros((1, 1), jnp.float32)
    for b in range(NSB):
        gb = goodf[:, b * 128:(b + 1) * 128]
        bb = badf[:, b * 128:(b + 1) * 128]
        posg.append(lax.dot_general(gb, lt, (((1,), (0,)), ((), ())),
                                    precision=_PREC,
                                    preferred_element_type=jnp.float32) + og)
        posb.append(lax.dot_general(bb, lt, (((1,), (0,)), ((), ())),
                                    precision=_PREC,
                                    preferred_element_type=jnp.float32) + ob)
        og = og + jnp.sum(gb, axis=1, keepdims=True)
        ob = ob + jnp.sum(bb, axis=1, keepdims=True)
    posg = jnp.concatenate(posg, axis=1)
    posb = jnp.concatenate(posb, axis=1)
    gc = jnp.minimum(og, float(POST_NMS_TOPN))

    dest = jnp.where(goodf > 0.5, posg,
                     jnp.where(badf > 0.5, gc + posb, 1e9))
    dest = jnp.where(dest < float(POST_NMS_TOPN), dest, 1e9)

    kcol = lax.broadcasted_iota(jnp.int32, (NOUT, 128), 0)
    desti = dest.astype(jnp.int32)
    acc = jnp.zeros((NOUT, 16), jnp.float32)
    for b in range(NSB):
        db = desti[:, b * 128:(b + 1) * 128]
        oh = jnp.where(db == kcol, 1.0, 0.0)
        blk = in_ref[b * 128:(b + 1) * 128, :]
        acc = acc + lax.dot_general(oh, blk, (((1,), (0,)), ((), ())),
                                    precision=_PREC,
                                    preferred_element_type=jnp.float32)
    out_ref[...] = acc


def kernel(scores, bbox_deltas, im_info):
    raw = jnp.transpose(scores[:, A:, :, :], (0, 2, 3, 1)).reshape(NB, 128)
    d = jnp.transpose(bbox_deltas, (0, 2, 3, 1)).reshape(-1, 4)
    dx = d[:, 0].reshape(NB, 128)
    dy = d[:, 1].reshape(NB, 128)
    dw = d[:, 2].reshape(NB, 128)
    dh = d[:, 3].reshape(NB, 128)
    anc = jnp.asarray(_ALL_ANCHORS)
    ax1 = anc[:, 0].reshape(NB, 128)
    ay1 = anc[:, 1].reshape(NB, 128)
    ax2 = anc[:, 2].reshape(NB, 128)
    ay2 = anc[:, 3].reshape(NB, 128)
    lims = jnp.concatenate([
        im_info[0:1, 1:2] - 1.0,
        im_info[0:1, 0:1] - 1.0,
        MIN_SIZE * im_info[0:1, 2:3],
        jnp.zeros((1, 125), jnp.float32)], axis=1)

    rows, rank = pl.pallas_call(
        _k1_body,
        out_shape=[jax.ShapeDtypeStruct((N, 16), jnp.float32),
                   jax.ShapeDtypeStruct((NB, 128), jnp.int32)],
        scratch_shapes=[pltpu.VMEM((NB, 128), jnp.float32)] * 5,
    )(ax1, ay1, ax2, ay2, dx, dy, dw, dh, raw, lims)

    sorted_rows = pl.pallas_call(
        _k2_body,
        out_shape=jax.ShapeDtypeStruct((NS, 16), jnp.float32),
    )(rows, rank)

    outp = pl.pallas_call(
        _k3_body,
        out_shape=jax.ShapeDtypeStruct((NOUT, 16), jnp.float32),
        scratch_shapes=[pltpu.VMEM((16, NS), jnp.float32),
                        pltpu.VMEM((128, 128), jnp.float32)],
    )(sorted_rows)

    boxes = outp[:POST_NMS_TOPN, 0:4]
    return jnp.concatenate([jnp.zeros((POST_NMS_TOPN, 1), jnp.float32), boxes],
                           axis=1)


# speedup vs baseline: 10.1919x; 1.4884x over previous
import functools

import numpy as np
import jax
import jax.numpy as jnp
from jax import lax
from jax.experimental import pallas as pl
from jax.experimental.pallas import tpu as pltpu
from jax.experimental.pallas import tpu_sc as plsc

FEATURE_STRIDE = 16
PRE_NMS_TOPN = 6000
POST_NMS_TOPN = 300
NMS_THRESH = 0.7
MIN_SIZE = 16.0
NEG = -1e9

H = W = 32
A = 9
N = H * W * A
NB = N // 128
NS = 6144
NSB = NS // 128
NOUT = 384
_PREC = lax.Precision.HIGHEST


def _gen_anchors():
    scales = np.array([8.0, 16.0, 32.0])
    ratios = np.array([0.5, 1.0, 2.0])
    base = np.array([1.0, 1.0, 16.0, 16.0]) - 1.0
    w = base[2] - base[0] + 1.0
    h = base[3] - base[1] + 1.0
    xc = base[0] + 0.5 * (w - 1.0)
    yc = base[1] + 0.5 * (h - 1.0)
    size = w * h
    ws = np.round(np.sqrt(size / ratios))
    hs = np.round(ws * ratios)

    def mk(ws, hs):
        ws = ws[:, None]
        hs = hs[:, None]
        return np.hstack((xc - 0.5 * (ws - 1.0), yc - 0.5 * (hs - 1.0),
                          xc + 0.5 * (ws - 1.0), yc + 0.5 * (hs - 1.0)))

    ra = mk(ws, hs)
    outs = []
    for i in range(ra.shape[0]):
        aw = ra[i, 2] - ra[i, 0] + 1.0
        ah = ra[i, 3] - ra[i, 1] + 1.0
        axc = ra[i, 0] + 0.5 * (aw - 1.0)
        ayc = ra[i, 1] + 0.5 * (ah - 1.0)
        ws2 = (aw * scales)[:, None]
        hs2 = (ah * scales)[:, None]
        outs.append(np.hstack((axc - 0.5 * (ws2 - 1.0), ayc - 0.5 * (hs2 - 1.0),
                               axc + 0.5 * (ws2 - 1.0), ayc + 0.5 * (hs2 - 1.0))))
    anchors = np.vstack(outs).astype(np.float32)
    sx, sy = np.meshgrid(np.arange(W) * FEATURE_STRIDE, np.arange(H) * FEATURE_STRIDE)
    shifts = np.stack([sx.ravel(), sy.ravel(), sx.ravel(), sy.ravel()], 1).astype(np.float32)
    return (anchors[None, :, :] + shifts[:, None, :]).reshape(-1, 4)


_ALL_ANCHORS = _gen_anchors()


def _eye(n):
    r = lax.broadcasted_iota(jnp.int32, (n, n), 0)
    c = lax.broadcasted_iota(jnp.int32, (n, n), 1)
    return jnp.where(r == c, 1.0, 0.0).astype(jnp.float32)


def _t(x):
    e = _eye(x.shape[1])
    return lax.dot_general(e, x, (((1,), (1,)), ((), ())),
                           precision=_PREC, preferred_element_type=jnp.float32)


def _k1_body(ax1, ay1, ax2, ay2, dx, dy, dw, dh, raw, lims,
             rows_ref, rank_ref, sc_ref, x1_ref, y1_ref, x2_ref, y2_ref):
    widths = ax2[...] - ax1[...] + 1.0
    heights = ay2[...] - ay1[...] + 1.0
    ctrx = ax1[...] + 0.5 * widths
    ctry = ay1[...] + 0.5 * heights
    pcx = dx[...] * widths + ctrx
    pcy = dy[...] * heights + ctry
    pw = jnp.exp(dw[...]) * widths
    ph = jnp.exp(dh[...]) * heights
    x1 = pcx - 0.5 * pw
    y1 = pcy - 0.5 * ph
    x2 = pcx + 0.5 * pw
    y2 = pcy + 0.5 * ph
    lv = lims[...]
    wmax = lv[0:1, 0:1]
    hmax = lv[0:1, 1:2]
    msz = lv[0:1, 2:3]
    x1 = jnp.minimum(jnp.maximum(x1, 0.0), wmax)
    y1 = jnp.minimum(jnp.maximum(y1, 0.0), hmax)
    x2 = jnp.minimum(jnp.maximum(x2, 0.0), wmax)
    y2 = jnp.minimum(jnp.maximum(y2, 0.0), hmax)
    ws = x2 - x1 + 1.0
    hs = y2 - y1 + 1.0
    valid = (ws >= msz) & (hs >= msz)
    sc = jnp.where(valid, raw[...], NEG)
    sc_ref[...] = sc
    x1_ref[...] = x1
    y1_ref[...] = y1
    x2_ref[...] = x2
    y2_ref[...] = y2

    lane = lax.broadcasted_iota(jnp.int32, (1, 128), 1)
    sub = lax.broadcasted_iota(jnp.int32, (128, 1), 0)

    def rows_body(ib, _):
        c5 = jnp.concatenate([
            x1_ref[pl.ds(ib, 1), :],
            y1_ref[pl.ds(ib, 1), :],
            x2_ref[pl.ds(ib, 1), :],
            y2_ref[pl.ds(ib, 1), :],
            sc_ref[pl.ds(ib, 1), :],
        ], axis=0)
        t = _t(c5)
        tp = jnp.concatenate([t, jnp.zeros((128, 123), jnp.float32)], axis=1)
        rows_ref[pl.ds(ib * 128, 128), :] = tp
        return 0

    lax.fori_loop(0, NB, rows_body, 0)

    ltm = lane < sub

    def rank_body(ib, _):
        srow_i = sc_ref[pl.ds(ib, 1), :]
        scol = _t(srow_i)

        def ge_body(jb, acc):
            srow = sc_ref[pl.ds(jb, 1), :]
            return acc + jnp.sum(jnp.where(srow >= scol, 1.0, 0.0),
                                 axis=1, keepdims=True)

        def gt_body(jb, acc):
            srow = sc_ref[pl.ds(jb, 1), :]
            return acc + jnp.sum(jnp.where(srow > scol, 1.0, 0.0),
                                 axis=1, keepdims=True)

        acc = lax.fori_loop(0, ib, ge_body, jnp.zeros((128, 1), jnp.float32))
        acc = lax.fori_loop(ib + 1, NB, gt_body, acc)
        diag = (srow_i > scol) | ((srow_i == scol) & ltm)
        acc = acc + jnp.sum(jnp.where(diag, 1.0, 0.0), axis=1, keepdims=True)
        rrow = lax.dot_general(acc, _eye(128), (((0,), (0,)), ((), ())),
                               precision=_PREC,
                               preferred_element_type=jnp.float32)
        rank_ref[pl.ds(ib, 1), :] = rrow.astype(jnp.int32)
        return 0

    lax.fori_loop(0, NB, rank_body, 0)


_SC_NW = 32
_SC_ROWS = N // _SC_NW
_SC_CHUNK = 96
_SC_NCH = _SC_ROWS // _SC_CHUNK


def _sc_scatter_body(rows_hbm, rank_hbm, out_hbm, idx_v, data_v, sem):
    wid = lax.axis_index("s") * 2 + lax.axis_index("c")
    base = wid * _SC_ROWS
    for ch in range(_SC_NCH):
        off = base + ch * _SC_CHUNK
        pltpu.sync_copy(rank_hbm.at[pl.ds(off, _SC_CHUNK)], idx_v)
        pltpu.sync_copy(rows_hbm.at[pl.ds(off, _SC_CHUNK), :], data_v)
        pltpu.async_copy(data_v, out_hbm.at[idx_v], sem).wait()


def _sc_scatter(rows, rank_flat):
    mesh = plsc.VectorSubcoreMesh(core_axis_name="c", subcore_axis_name="s")
    f = functools.partial(
        pl.kernel,
        mesh=mesh,
        out_type=jax.ShapeDtypeStruct((N, 128), jnp.float32),
        scratch_types=[
            pltpu.VMEM((_SC_CHUNK,), jnp.int32),
            pltpu.VMEM((_SC_CHUNK, 128), jnp.float32),
            pltpu.SemaphoreType.DMA,
        ],
    )(_sc_scatter_body)
    return f(rows, rank_flat)


def _k3_body(in_ref, out_ref, rowsT_ref, w_ref):
    e16 = _eye(16)
    for b in range(NSB):
        blk = in_ref[b * 128:(b + 1) * 128, 0:16]
        rowsT_ref[:, b * 128:(b + 1) * 128] = lax.dot_general(
            e16, blk, (((1,), (1,)), ((), ())),
            precision=_PREC, preferred_element_type=jnp.float32)

    x1r = rowsT_ref[0:1, :]
    y1r = rowsT_ref[1:2, :]
    x2r = rowsT_ref[2:3, :]
    y2r = rowsT_ref[3:4, :]
    areas_row = (x2r - x1r + 1.0) * (y2r - y1r + 1.0)

    lane1 = lax.broadcasted_iota(jnp.int32, (1, 128), 1)
    slot_full = lax.broadcasted_iota(jnp.int32, (1, NS), 1)
    supp = jnp.zeros((1, NS), jnp.float32)
    keepall = []

    for b in range(NSB):
        base = b * 128
        blk = in_ref[base:base + 128, 0:16]
        bx1 = blk[:, 0:1]
        by1 = blk[:, 1:2]
        bx2 = blk[:, 2:3]
        by2 = blk[:, 3:4]
        areac = (bx2 - bx1 + 1.0) * (by2 - by1 + 1.0)
        xx1 = jnp.maximum(bx1, x1r[:, base:])
        yy1 = jnp.maximum(by1, y1r[:, base:])
        xx2 = jnp.minimum(bx2, x2r[:, base:])
        yy2 = jnp.minimum(by2, y2r[:, base:])
        iw = jnp.maximum(xx2 - xx1 + 1.0, 0.0)
        ih = jnp.maximum(yy2 - yy1 + 1.0, 0.0)
        inter = iw * ih
        iou = inter / (areac + areas_row[:, base:] - inter)
        w_ref[...] = iou[:, 0:128]

        validb = jnp.where(base + lane1 < PRE_NMS_TOPN, 1.0, 0.0)
        keep0 = validb * (1.0 - supp[:, base:base + 128])

        def greedy(i, keep):
            wrow = w_ref[pl.ds(i, 1), :]
            ki = jnp.max(jnp.where(lane1 == i, keep, 0.0))
            kill = (wrow > NMS_THRESH) & (lane1 > i) & (ki > 0.5)
            return jnp.where(kill, 0.0, keep)

        keep = lax.fori_loop(0, 128, greedy, keep0)
        keepall.append(keep)

        kc = _t(keep)
        supmat = jnp.where((iou > NMS_THRESH) & (kc > 0.5), 1.0, 0.0)
        supnew = jnp.max(supmat, axis=0, keepdims=True)
        if b < NSB - 1:
            tail = jnp.maximum(supp[:, base + 128:], supnew[:, 128:])
            supp = jnp.concatenate([supp[:, :base + 128], tail], axis=1)

    keepall = jnp.concatenate(keepall, axis=1)

    sc_row = rowsT_ref[4:5, :]
    slot_ok = jnp.where(slot_full < PRE_NMS_TOPN, 1.0, 0.0)
    goodf = keepall * jnp.where(sc_row != NEG, 1.0, 0.0) * slot_ok
    badf = (1.0 - goodf) * slot_ok

    lt = jnp.where(lax.broadcasted_iota(jnp.int32, (128, 128), 0) <
                   lax.broadcasted_iota(jnp.int32, (128, 128), 1), 1.0, 0.0)
    posg, posb = [], []
    og = jnp.zeros((1, 1), jnp.float32)
    ob = jnp.zeros((1, 1), jnp.float32)
    for b in range(NSB):
        gb = goodf[:, b * 128:(b + 1) * 128]
        bb = badf[:, b * 128:(b + 1) * 128]
        posg.append(lax.dot_general(gb, lt, (((1,), (0,)), ((), ())),
                                    precision=_PREC,
                                    preferred_element_type=jnp.float32) + og)
        posb.append(lax.dot_general(bb, lt, (((1,), (0,)), ((), ())),
                                    precision=_PREC,
                                    preferred_element_type=jnp.float32) + ob)
        og = og + jnp.sum(gb, axis=1, keepdims=True)
        ob = ob + jnp.sum(bb, axis=1, keepdims=True)
    posg = jnp.concatenate(posg, axis=1)
    posb = jnp.concatenate(posb, axis=1)
    gc = jnp.minimum(og, float(POST_NMS_TOPN))

    dest = jnp.where(goodf > 0.5, posg,
                     jnp.where(badf > 0.5, gc + posb, 1e9))
    dest = jnp.where(dest < float(POST_NMS_TOPN), dest, 1e9)

    kcol = lax.broadcasted_iota(jnp.int32, (NOUT, 128), 0)
    desti = dest.astype(jnp.int32)
    acc = jnp.zeros((NOUT, 16), jnp.float32)
    for b in range(NSB):
        db = desti[:, b * 128:(b + 1) * 128]
        oh = jnp.where(db == kcol, 1.0, 0.0)
        blk = in_ref[b * 128:(b + 1) * 128, 0:16]
        acc = acc + lax.dot_general(oh, blk, (((1,), (0,)), ((), ())),
                                    precision=_PREC,
                                    preferred_element_type=jnp.float32)
    out_ref[...] = acc


def kernel(scores, bbox_deltas, im_info):
    raw = jnp.transpose(scores[:, A:, :, :], (0, 2, 3, 1)).reshape(NB, 128)
    d = jnp.transpose(bbox_deltas, (0, 2, 3, 1)).reshape(-1, 4)
    dx = d[:, 0].reshape(NB, 128)
    dy = d[:, 1].reshape(NB, 128)
    dw = d[:, 2].reshape(NB, 128)
    dh = d[:, 3].reshape(NB, 128)
    anc = jnp.asarray(_ALL_ANCHORS)
    ax1 = anc[:, 0].reshape(NB, 128)
    ay1 = anc[:, 1].reshape(NB, 128)
    ax2 = anc[:, 2].reshape(NB, 128)
    ay2 = anc[:, 3].reshape(NB, 128)
    lims = jnp.concatenate([
        im_info[0:1, 1:2] - 1.0,
        im_info[0:1, 0:1] - 1.0,
        MIN_SIZE * im_info[0:1, 2:3],
        jnp.zeros((1, 125), jnp.float32)], axis=1)

    rows, rank = pl.pallas_call(
        _k1_body,
        out_shape=[jax.ShapeDtypeStruct((N, 128), jnp.float32),
                   jax.ShapeDtypeStruct((NB, 128), jnp.int32)],
        scratch_shapes=[pltpu.VMEM((NB, 128), jnp.float32)] * 5,
    )(ax1, ay1, ax2, ay2, dx, dy, dw, dh, raw, lims)

    sorted_perm = _sc_scatter(rows, rank.reshape(N))
    sorted_rows = sorted_perm[:NS, :]

    outp = pl.pallas_call(
        _k3_body,
        out_shape=jax.ShapeDtypeStruct((NOUT, 16), jnp.float32),
        scratch_shapes=[pltpu.VMEM((16, NS), jnp.float32),
                        pltpu.VMEM((128, 128), jnp.float32)],
    )(sorted_rows)

    boxes = outp[:POST_NMS_TOPN, 0:4]
    return jnp.concatenate([jnp.zeros((POST_NMS_TOPN, 1), jnp.float32), boxes],
                           axis=1)


# speedup vs baseline: 12.6423x; 1.2404x over previous
import functools

import numpy as np
import jax
import jax.numpy as jnp
from jax import lax
from jax.experimental import pallas as pl
from jax.experimental.pallas import tpu as pltpu
from jax.experimental.pallas import tpu_sc as plsc

FEATURE_STRIDE = 16
PRE_NMS_TOPN = 6000
POST_NMS_TOPN = 300
NMS_THRESH = 0.7
MIN_SIZE = 16.0
NEG = -1e9

H = W = 32
A = 9
N = H * W * A
NB = N // 128
NS = 6144
NSB = NS // 128
NOUT = 384
_PREC = lax.Precision.HIGHEST


def _gen_anchors():
    scales = np.array([8.0, 16.0, 32.0])
    ratios = np.array([0.5, 1.0, 2.0])
    base = np.array([1.0, 1.0, 16.0, 16.0]) - 1.0
    w = base[2] - base[0] + 1.0
    h = base[3] - base[1] + 1.0
    xc = base[0] + 0.5 * (w - 1.0)
    yc = base[1] + 0.5 * (h - 1.0)
    size = w * h
    ws = np.round(np.sqrt(size / ratios))
    hs = np.round(ws * ratios)

    def mk(ws, hs):
        ws = ws[:, None]
        hs = hs[:, None]
        return np.hstack((xc - 0.5 * (ws - 1.0), yc - 0.5 * (hs - 1.0),
                          xc + 0.5 * (ws - 1.0), yc + 0.5 * (hs - 1.0)))

    ra = mk(ws, hs)
    outs = []
    for i in range(ra.shape[0]):
        aw = ra[i, 2] - ra[i, 0] + 1.0
        ah = ra[i, 3] - ra[i, 1] + 1.0
        axc = ra[i, 0] + 0.5 * (aw - 1.0)
        ayc = ra[i, 1] + 0.5 * (ah - 1.0)
        ws2 = (aw * scales)[:, None]
        hs2 = (ah * scales)[:, None]
        outs.append(np.hstack((axc - 0.5 * (ws2 - 1.0), ayc - 0.5 * (hs2 - 1.0),
                               axc + 0.5 * (ws2 - 1.0), ayc + 0.5 * (hs2 - 1.0))))
    anchors = np.vstack(outs).astype(np.float32)
    sx, sy = np.meshgrid(np.arange(W) * FEATURE_STRIDE, np.arange(H) * FEATURE_STRIDE)
    shifts = np.stack([sx.ravel(), sy.ravel(), sx.ravel(), sy.ravel()], 1).astype(np.float32)
    return (anchors[None, :, :] + shifts[:, None, :]).reshape(-1, 4)


_ALL_ANCHORS = _gen_anchors()


def _eye(n):
    r = lax.broadcasted_iota(jnp.int32, (n, n), 0)
    c = lax.broadcasted_iota(jnp.int32, (n, n), 1)
    return jnp.where(r == c, 1.0, 0.0).astype(jnp.float32)


def _t(x):
    e = _eye(x.shape[1])
    return lax.dot_general(e, x, (((1,), (1,)), ((), ())),
                           precision=_PREC, preferred_element_type=jnp.float32)


def _k1_body(ax1, ay1, ax2, ay2, dx, dy, dw, dh, raw, lims,
             rows_ref, rank_ref, sc_ref, x1_ref, y1_ref, x2_ref, y2_ref):
    widths = ax2[...] - ax1[...] + 1.0
    heights = ay2[...] - ay1[...] + 1.0
    ctrx = ax1[...] + 0.5 * widths
    ctry = ay1[...] + 0.5 * heights
    pcx = dx[...] * widths + ctrx
    pcy = dy[...] * heights + ctry
    pw = jnp.exp(dw[...]) * widths
    ph = jnp.exp(dh[...]) * heights
    x1 = pcx - 0.5 * pw
    y1 = pcy - 0.5 * ph
    x2 = pcx + 0.5 * pw
    y2 = pcy + 0.5 * ph
    lv = lims[...]
    wmax = lv[0:1, 0:1]
    hmax = lv[0:1, 1:2]
    msz = lv[0:1, 2:3]
    x1 = jnp.minimum(jnp.maximum(x1, 0.0), wmax)
    y1 = jnp.minimum(jnp.maximum(y1, 0.0), hmax)
    x2 = jnp.minimum(jnp.maximum(x2, 0.0), wmax)
    y2 = jnp.minimum(jnp.maximum(y2, 0.0), hmax)
    ws = x2 - x1 + 1.0
    hs = y2 - y1 + 1.0
    valid = (ws >= msz) & (hs >= msz)
    sc = jnp.where(valid, raw[...], NEG)
    sc_ref[...] = sc
    x1_ref[...] = x1
    y1_ref[...] = y1
    x2_ref[...] = x2
    y2_ref[...] = y2

    lane = lax.broadcasted_iota(jnp.int32, (1, 128), 1)
    sub = lax.broadcasted_iota(jnp.int32, (128, 1), 0)

    def rows_body(ib, _):
        c5 = jnp.concatenate([
            x1_ref[pl.ds(ib, 1), :],
            y1_ref[pl.ds(ib, 1), :],
            x2_ref[pl.ds(ib, 1), :],
            y2_ref[pl.ds(ib, 1), :],
            sc_ref[pl.ds(ib, 1), :],
        ], axis=0)
        t = _t(c5)
        tp = jnp.concatenate([t, jnp.zeros((128, 123), jnp.float32)], axis=1)
        rows_ref[pl.ds(ib * 128, 128), :] = tp
        return 0

    lax.fori_loop(0, NB, rows_body, 0)

    ltm = lane < sub

    def rank_body(ib, _):
        srow_i = sc_ref[pl.ds(ib, 1), :]
        scol = _t(srow_i)

        def ge_body(jb, acc):
            srow = sc_ref[pl.ds(jb, 1), :]
            return acc + jnp.sum(jnp.where(srow >= scol, 1.0, 0.0),
                                 axis=1, keepdims=True)

        def gt_body(jb, acc):
            srow = sc_ref[pl.ds(jb, 1), :]
            return acc + jnp.sum(jnp.where(srow > scol, 1.0, 0.0),
                                 axis=1, keepdims=True)

        acc = lax.fori_loop(0, ib, ge_body, jnp.zeros((128, 1), jnp.float32))
        acc = lax.fori_loop(ib + 1, NB, gt_body, acc)
        diag = (srow_i > scol) | ((srow_i == scol) & ltm)
        acc = acc + jnp.sum(jnp.where(diag, 1.0, 0.0), axis=1, keepdims=True)
        rrow = lax.dot_general(acc, _eye(128), (((0,), (0,)), ((), ())),
                               precision=_PREC,
                               preferred_element_type=jnp.float32)
        rank_ref[pl.ds(ib, 1), :] = rrow.astype(jnp.int32)
        return 0

    lax.fori_loop(0, NB, rank_body, 0)


_SC_NW = 32
_SC_ROWS = N // _SC_NW
_SC_CHUNK = 96
_SC_NCH = _SC_ROWS // _SC_CHUNK


def _sc_scatter_body(rows_hbm, rank_hbm, out_hbm, idx_v, data_v, sem):
    wid = lax.axis_index("s") * 2 + lax.axis_index("c")
    base = wid * _SC_ROWS
    for ch in range(_SC_NCH):
        off = base + ch * _SC_CHUNK
        pltpu.sync_copy(rank_hbm.at[pl.ds(off, _SC_CHUNK)], idx_v)
        pltpu.sync_copy(rows_hbm.at[pl.ds(off, _SC_CHUNK), :], data_v)
        pltpu.async_copy(data_v, out_hbm.at[idx_v], sem).wait()


def _sc_scatter(rows, rank_flat):
    mesh = plsc.VectorSubcoreMesh(core_axis_name="c", subcore_axis_name="s")
    f = functools.partial(
        pl.kernel,
        mesh=mesh,
        out_type=jax.ShapeDtypeStruct((N, 128), jnp.float32),
        scratch_types=[
            pltpu.VMEM((_SC_CHUNK,), jnp.int32),
            pltpu.VMEM((_SC_CHUNK, 128), jnp.float32),
            pltpu.SemaphoreType.DMA,
        ],
    )(_sc_scatter_body)
    return f(rows, rank_flat)


def _k3_body(in_ref, out_ref, rowsT_ref, w_ref):
    e16 = _eye(16)
    for b in range(NSB):
        blk = in_ref[b * 128:(b + 1) * 128, 0:16]
        rowsT_ref[:, b * 128:(b + 1) * 128] = lax.dot_general(
            e16, blk, (((1,), (1,)), ((), ())),
            precision=_PREC, preferred_element_type=jnp.float32)

    x1r = rowsT_ref[0:1, :]
    y1r = rowsT_ref[1:2, :]
    x2r = rowsT_ref[2:3, :]
    y2r = rowsT_ref[3:4, :]
    areas_row = (x2r - x1r + 1.0) * (y2r - y1r + 1.0)

    lane1 = lax.broadcasted_iota(jnp.int32, (1, 128), 1)
    slot_full = lax.broadcasted_iota(jnp.int32, (1, NS), 1)
    supp = jnp.zeros((1, NS), jnp.float32)
    keepall = []

    for b in range(NSB):
        base = b * 128
        blk = in_ref[base:base + 128, 0:16]
        bx1 = blk[:, 0:1]
        by1 = blk[:, 1:2]
        bx2 = blk[:, 2:3]
        by2 = blk[:, 3:4]
        areac = (bx2 - bx1 + 1.0) * (by2 - by1 + 1.0)
        xx1 = jnp.maximum(bx1, x1r[:, base:])
        yy1 = jnp.maximum(by1, y1r[:, base:])
        xx2 = jnp.minimum(bx2, x2r[:, base:])
        yy2 = jnp.minimum(by2, y2r[:, base:])
        iw = jnp.maximum(xx2 - xx1 + 1.0, 0.0)
        ih = jnp.maximum(yy2 - yy1 + 1.0, 0.0)
        inter = iw * ih
        iou = inter / (areac + areas_row[:, base:] - inter)
        w_ref[...] = iou[:, 0:128]

        validb = jnp.where(base + lane1 < PRE_NMS_TOPN, 1.0, 0.0)
        keep0 = validb * (1.0 - supp[:, base:base + 128])

        def g_cond(state):
            alive, _ = state
            return jnp.max(alive) > 0.5

        def g_body(state):
            alive, kept = state
            i = jnp.min(jnp.where(alive > 0.5, lane1, 128))
            wrow = w_ref[pl.ds(i, 1), :]
            kept = jnp.where(lane1 == i, 1.0, kept)
            alive = jnp.where((lane1 > i) & ~(wrow > NMS_THRESH), alive, 0.0)
            return alive, kept

        _, keep = lax.while_loop(g_cond, g_body,
                                 (keep0, jnp.zeros((1, 128), jnp.float32)))
        keepall.append(keep)

        kc = _t(keep)
        supmat = jnp.where((iou > NMS_THRESH) & (kc > 0.5), 1.0, 0.0)
        supnew = jnp.max(supmat, axis=0, keepdims=True)
        if b < NSB - 1:
            tail = jnp.maximum(supp[:, base + 128:], supnew[:, 128:])
            supp = jnp.concatenate([supp[:, :base + 128], tail], axis=1)

    keepall = jnp.concatenate(keepall, axis=1)

    sc_row = rowsT_ref[4:5, :]
    slot_ok = jnp.where(slot_full < PRE_NMS_TOPN, 1.0, 0.0)
    goodf = keepall * jnp.where(sc_row != NEG, 1.0, 0.0) * slot_ok
    badf = (1.0 - goodf) * slot_ok

    lt = jnp.where(lax.broadcasted_iota(jnp.int32, (128, 128), 0) <
                   lax.broadcasted_iota(jnp.int32, (128, 128), 1), 1.0, 0.0)
    posg, posb = [], []
    og = jnp.zeros((1, 1), jnp.float32)
    ob = jnp.zeros((1, 1), jnp.float32)
    for b in range(NSB):
        gb = goodf[:, b * 128:(b + 1) * 128]
        bb = badf[:, b * 128:(b + 1) * 128]
        posg.append(lax.dot_general(gb, lt, (((1,), (0,)), ((), ())),
                                    precision=_PREC,
                                    preferred_element_type=jnp.float32) + og)
        posb.append(lax.dot_general(bb, lt, (((1,), (0,)), ((), ())),
                                    precision=_PREC,
                                    preferred_element_type=jnp.float32) + ob)
        og = og + jnp.sum(gb, axis=1, keepdims=True)
        ob = ob + jnp.sum(bb, axis=1, keepdims=True)
    posg = jnp.concatenate(posg, axis=1)
    posb = jnp.concatenate(posb, axis=1)
    gc = jnp.minimum(og, float(POST_NMS_TOPN))

    dest = jnp.where(goodf > 0.5, posg,
                     jnp.where(badf > 0.5, gc + posb, 1e9))
    dest = jnp.where(dest < float(POST_NMS_TOPN), dest, 1e9)

    kcol = lax.broadcasted_iota(jnp.int32, (NOUT, 128), 0)
    desti = dest.astype(jnp.int32)
    acc = jnp.zeros((NOUT, 16), jnp.float32)
    for b in range(NSB):
        db = desti[:, b * 128:(b + 1) * 128]
        oh = jnp.where(db == kcol, 1.0, 0.0)
        blk = in_ref[b * 128:(b + 1) * 128, 0:16]
        acc = acc + lax.dot_general(oh, blk, (((1,), (0,)), ((), ())),
                                    precision=_PREC,
                                    preferred_element_type=jnp.float32)
    out_ref[...] = acc


def kernel(scores, bbox_deltas, im_info):
    raw = jnp.transpose(scores[:, A:, :, :], (0, 2, 3, 1)).reshape(NB, 128)
    d = jnp.transpose(bbox_deltas, (0, 2, 3, 1)).reshape(-1, 4)
    dx = d[:, 0].reshape(NB, 128)
    dy = d[:, 1].reshape(NB, 128)
    dw = d[:, 2].reshape(NB, 128)
    dh = d[:, 3].reshape(NB, 128)
    anc = jnp.asarray(_ALL_ANCHORS)
    ax1 = anc[:, 0].reshape(NB, 128)
    ay1 = anc[:, 1].reshape(NB, 128)
    ax2 = anc[:, 2].reshape(NB, 128)
    ay2 = anc[:, 3].reshape(NB, 128)
    lims = jnp.concatenate([
        im_info[0:1, 1:2] - 1.0,
        im_info[0:1, 0:1] - 1.0,
        MIN_SIZE * im_info[0:1, 2:3],
        jnp.zeros((1, 125), jnp.float32)], axis=1)

    rows, rank = pl.pallas_call(
        _k1_body,
        out_shape=[jax.ShapeDtypeStruct((N, 128), jnp.float32),
                   jax.ShapeDtypeStruct((NB, 128), jnp.int32)],
        scratch_shapes=[pltpu.VMEM((NB, 128), jnp.float32)] * 5,
    )(ax1, ay1, ax2, ay2, dx, dy, dw, dh, raw, lims)

    sorted_perm = _sc_scatter(rows, rank.reshape(N))
    sorted_rows = sorted_perm[:NS, :]

    outp = pl.pallas_call(
        _k3_body,
        out_shape=jax.ShapeDtypeStruct((NOUT, 16), jnp.float32),
        scratch_shapes=[pltpu.VMEM((16, NS), jnp.float32),
                        pltpu.VMEM((128, 128), jnp.float32)],
    )(sorted_rows)

    boxes = outp[:POST_NMS_TOPN, 0:4]
    return jnp.concatenate([jnp.zeros((POST_NMS_TOPN, 1), jnp.float32), boxes],
                           axis=1)


# speedup vs baseline: 19.4752x; 1.5405x over previous
import functools

import numpy as np
import jax
import jax.numpy as jnp
from jax import lax
from jax.experimental import pallas as pl
from jax.experimental.pallas import tpu as pltpu
from jax.experimental.pallas import tpu_sc as plsc

FEATURE_STRIDE = 16
PRE_NMS_TOPN = 6000
POST_NMS_TOPN = 300
NMS_THRESH = 0.7
MIN_SIZE = 16.0
NEG = -1e9

H = W = 32
A = 9
N = H * W * A
NB = N // 128
NS = 6144
NSB = NS // 128
NOUT = 384
_PREC = lax.Precision.HIGHEST


def _gen_anchors():
    scales = np.array([8.0, 16.0, 32.0])
    ratios = np.array([0.5, 1.0, 2.0])
    base = np.array([1.0, 1.0, 16.0, 16.0]) - 1.0
    w = base[2] - base[0] + 1.0
    h = base[3] - base[1] + 1.0
    xc = base[0] + 0.5 * (w - 1.0)
    yc = base[1] + 0.5 * (h - 1.0)
    size = w * h
    ws = np.round(np.sqrt(size / ratios))
    hs = np.round(ws * ratios)

    def mk(ws, hs):
        ws = ws[:, None]
        hs = hs[:, None]
        return np.hstack((xc - 0.5 * (ws - 1.0), yc - 0.5 * (hs - 1.0),
                          xc + 0.5 * (ws - 1.0), yc + 0.5 * (hs - 1.0)))

    ra = mk(ws, hs)
    outs = []
    for i in range(ra.shape[0]):
        aw = ra[i, 2] - ra[i, 0] + 1.0
        ah = ra[i, 3] - ra[i, 1] + 1.0
        axc = ra[i, 0] + 0.5 * (aw - 1.0)
        ayc = ra[i, 1] + 0.5 * (ah - 1.0)
        ws2 = (aw * scales)[:, None]
        hs2 = (ah * scales)[:, None]
        outs.append(np.hstack((axc - 0.5 * (ws2 - 1.0), ayc - 0.5 * (hs2 - 1.0),
                               axc + 0.5 * (ws2 - 1.0), ayc + 0.5 * (hs2 - 1.0))))
    anchors = np.vstack(outs).astype(np.float32)
    sx, sy = np.meshgrid(np.arange(W) * FEATURE_STRIDE, np.arange(H) * FEATURE_STRIDE)
    shifts = np.stack([sx.ravel(), sy.ravel(), sx.ravel(), sy.ravel()], 1).astype(np.float32)
    return (anchors[None, :, :] + shifts[:, None, :]).reshape(-1, 4)


_ALL_ANCHORS = _gen_anchors()


def _eye(n):
    r = lax.broadcasted_iota(jnp.int32, (n, n), 0)
    c = lax.broadcasted_iota(jnp.int32, (n, n), 1)
    return jnp.where(r == c, 1.0, 0.0).astype(jnp.float32)


def _t(x):
    e = _eye(x.shape[1])
    return lax.dot_general(e, x, (((1,), (1,)), ((), ())),
                           precision=_PREC, preferred_element_type=jnp.float32)


def _k1_body(ax1, ay1, ax2, ay2, dx, dy, dw, dh, raw, lims,
             rows_ref, rank_ref, sc_ref, x1_ref, y1_ref, x2_ref, y2_ref,
             rka_ref):
    widths = ax2[...] - ax1[...] + 1.0
    heights = ay2[...] - ay1[...] + 1.0
    ctrx = ax1[...] + 0.5 * widths
    ctry = ay1[...] + 0.5 * heights
    pcx = dx[...] * widths + ctrx
    pcy = dy[...] * heights + ctry
    pw = jnp.exp(dw[...]) * widths
    ph = jnp.exp(dh[...]) * heights
    x1 = pcx - 0.5 * pw
    y1 = pcy - 0.5 * ph
    x2 = pcx + 0.5 * pw
    y2 = pcy + 0.5 * ph
    lv = lims[...]
    wmax = lv[0:1, 0:1]
    hmax = lv[0:1, 1:2]
    msz = lv[0:1, 2:3]
    x1 = jnp.minimum(jnp.maximum(x1, 0.0), wmax)
    y1 = jnp.minimum(jnp.maximum(y1, 0.0), hmax)
    x2 = jnp.minimum(jnp.maximum(x2, 0.0), wmax)
    y2 = jnp.minimum(jnp.maximum(y2, 0.0), hmax)
    ws = x2 - x1 + 1.0
    hs = y2 - y1 + 1.0
    valid = (ws >= msz) & (hs >= msz)
    sc = jnp.where(valid, raw[...], NEG)
    sc_ref[...] = sc
    x1_ref[...] = x1
    y1_ref[...] = y1
    x2_ref[...] = x2
    y2_ref[...] = y2

    lane = lax.broadcasted_iota(jnp.int32, (1, 128), 1)
    sub = lax.broadcasted_iota(jnp.int32, (128, 1), 0)

    def rows_body(ib, _):
        c5 = jnp.concatenate([
            x1_ref[pl.ds(ib, 1), :],
            y1_ref[pl.ds(ib, 1), :],
            x2_ref[pl.ds(ib, 1), :],
            y2_ref[pl.ds(ib, 1), :],
            sc_ref[pl.ds(ib, 1), :],
        ], axis=0)
        t = _t(c5)
        tp = jnp.concatenate([t, jnp.zeros((128, 123), jnp.float32)], axis=1)
        rows_ref[pl.ds(ib * 128, 128), :] = tp
        return 0

    lax.fori_loop(0, NB, rows_body, 0)

    ltm = lane < sub
    rka_ref[...] = jnp.zeros((NB, 128), jnp.float32)

    def rank_body(ib, _):
        srow_i = sc_ref[pl.ds(ib, 1), :]
        scol = _t(srow_i)

        def pair_body(jb, acc):
            srow = sc_ref[pl.ds(jb, 1), :]
            m = jnp.where(srow > scol, 1.0, 0.0)
            colsum = jnp.sum(m, axis=0, keepdims=True)
            rka_ref[pl.ds(jb, 1), :] = (rka_ref[pl.ds(jb, 1), :]
                                        + (128.0 - colsum))
            return acc + m

        acc = lax.fori_loop(ib + 1, NB, pair_body,
                            jnp.zeros((128, 128), jnp.float32))
        diag = (srow_i > scol) | ((srow_i == scol) & ltm)
        acc = acc + jnp.where(diag, 1.0, 0.0)
        rowsum = jnp.sum(acc, axis=1, keepdims=True)
        rrow = lax.dot_general(rowsum, _eye(128), (((0,), (0,)), ((), ())),
                               precision=_PREC,
                               preferred_element_type=jnp.float32)
        rka_ref[pl.ds(ib, 1), :] = rka_ref[pl.ds(ib, 1), :] + rrow
        return 0

    lax.fori_loop(0, NB, rank_body, 0)
    rank_ref[...] = rka_ref[...].astype(jnp.int32)


_SC_NW = 32
_SC_ROWS = N // _SC_NW
_SC_CHUNK = 96
_SC_NCH = _SC_ROWS // _SC_CHUNK


def _sc_scatter_body(rows_hbm, rank_hbm, out_hbm, idx_v, data_v, sem):
    wid = lax.axis_index("s") * 2 + lax.axis_index("c")
    base = wid * _SC_ROWS
    for ch in range(_SC_NCH):
        off = base + ch * _SC_CHUNK
        pltpu.sync_copy(rank_hbm.at[pl.ds(off, _SC_CHUNK)], idx_v)
        pltpu.sync_copy(rows_hbm.at[pl.ds(off, _SC_CHUNK), :], data_v)
        pltpu.async_copy(data_v, out_hbm.at[idx_v], sem).wait()


def _sc_scatter(rows, rank_flat):
    mesh = plsc.VectorSubcoreMesh(core_axis_name="c", subcore_axis_name="s")
    f = functools.partial(
        pl.kernel,
        mesh=mesh,
        out_type=jax.ShapeDtypeStruct((N, 128), jnp.float32),
        scratch_types=[
            pltpu.VMEM((_SC_CHUNK,), jnp.int32),
            pltpu.VMEM((_SC_CHUNK, 128), jnp.float32),
            pltpu.SemaphoreType.DMA,
        ],
    )(_sc_scatter_body)
    return f(rows, rank_flat)


def _k3_body(in_ref, out_ref, rowsT_ref, w_ref):
    e16 = _eye(16)
    for b in range(NSB):
        blk = in_ref[b * 128:(b + 1) * 128, 0:16]
        rowsT_ref[:, b * 128:(b + 1) * 128] = lax.dot_general(
            e16, blk, (((1,), (1,)), ((), ())),
            precision=_PREC, preferred_element_type=jnp.float32)

    x1r = rowsT_ref[0:1, :]
    y1r = rowsT_ref[1:2, :]
    x2r = rowsT_ref[2:3, :]
    y2r = rowsT_ref[3:4, :]
    areas_row = (x2r - x1r + 1.0) * (y2r - y1r + 1.0)

    lane1 = lax.broadcasted_iota(jnp.int32, (1, 128), 1)
    slot_full = lax.broadcasted_iota(jnp.int32, (1, NS), 1)
    supp = jnp.zeros((1, NS), jnp.float32)
    keepall = []

    for b in range(NSB):
        base = b * 128
        blk = in_ref[base:base + 128, 0:16]
        bx1 = blk[:, 0:1]
        by1 = blk[:, 1:2]
        bx2 = blk[:, 2:3]
        by2 = blk[:, 3:4]
        areac = (bx2 - bx1 + 1.0) * (by2 - by1 + 1.0)
        xx1 = jnp.maximum(bx1, x1r[:, base:])
        yy1 = jnp.maximum(by1, y1r[:, base:])
        xx2 = jnp.minimum(bx2, x2r[:, base:])
        yy2 = jnp.minimum(by2, y2r[:, base:])
        iw = jnp.maximum(xx2 - xx1 + 1.0, 0.0)
        ih = jnp.maximum(yy2 - yy1 + 1.0, 0.0)
        inter = iw * ih
        iou = inter / (areac + areas_row[:, base:] - inter)
        w_ref[...] = iou[:, 0:128]

        validb = jnp.where(base + lane1 < PRE_NMS_TOPN, 1.0, 0.0)
        keep0 = validb * (1.0 - supp[:, base:base + 128])

        def g_cond(state):
            alive, _ = state
            return jnp.max(alive) > 0.5

        def g_body(state):
            alive, kept = state
            i = jnp.min(jnp.where(alive > 0.5, lane1, 128))
            wrow = w_ref[pl.ds(i, 1), :]
            kept = jnp.where(lane1 == i, 1.0, kept)
            alive = jnp.where((lane1 > i) & ~(wrow > NMS_THRESH), alive, 0.0)
            return alive, kept

        _, keep = lax.while_loop(g_cond, g_body,
                                 (keep0, jnp.zeros((1, 128), jnp.float32)))
        keepall.append(keep)

        kc = _t(keep)
        supmat = jnp.where((iou > NMS_THRESH) & (kc > 0.5), 1.0, 0.0)
        supnew = jnp.max(supmat, axis=0, keepdims=True)
        if b < NSB - 1:
            tail = jnp.maximum(supp[:, base + 128:], supnew[:, 128:])
            supp = jnp.concatenate([supp[:, :base + 128], tail], axis=1)

    keepall = jnp.concatenate(keepall, axis=1)

    sc_row = rowsT_ref[4:5, :]
    slot_ok = jnp.where(slot_full < PRE_NMS_TOPN, 1.0, 0.0)
    goodf = keepall * jnp.where(sc_row != NEG, 1.0, 0.0) * slot_ok
    badf = (1.0 - goodf) * slot_ok

    lt = jnp.where(lax.broadcasted_iota(jnp.int32, (128, 128), 0) <
                   lax.broadcasted_iota(jnp.int32, (128, 128), 1), 1.0, 0.0)
    posg, posb = [], []
    og = jnp.zeros((1, 1), jnp.float32)
    ob = jnp.zeros((1, 1), jnp.float32)
    for b in range(NSB):
        gb = goodf[:, b * 128:(b + 1) * 128]
        bb = badf[:, b * 128:(b + 1) * 128]
        posg.append(lax.dot_general(gb, lt, (((1,), (0,)), ((), ())),
                                    precision=_PREC,
                                    preferred_element_type=jnp.float32) + og)
        posb.append(lax.dot_general(bb, lt, (((1,), (0,)), ((), ())),
                                    precision=_PREC,
                                    preferred_element_type=jnp.float32) + ob)
        og = og + jnp.sum(gb, axis=1, keepdims=True)
        ob = ob + jnp.sum(bb, axis=1, keepdims=True)
    posg = jnp.concatenate(posg, axis=1)
    posb = jnp.concatenate(posb, axis=1)
    gc = jnp.minimum(og, float(POST_NMS_TOPN))

    dest = jnp.where(goodf > 0.5, posg,
                     jnp.where(badf > 0.5, gc + posb, 1e9))
    dest = jnp.where(dest < float(POST_NMS_TOPN), dest, 1e9)

    kcol = lax.broadcasted_iota(jnp.int32, (NOUT, 128), 0)
    desti = dest.astype(jnp.int32)
    acc = jnp.zeros((NOUT, 16), jnp.float32)
    for b in range(NSB):
        db = desti[:, b * 128:(b + 1) * 128]
        oh = jnp.where(db == kcol, 1.0, 0.0)
        blk = in_ref[b * 128:(b + 1) * 128, 0:16]
        acc = acc + lax.dot_general(oh, blk, (((1,), (0,)), ((), ())),
                                    precision=_PREC,
                                    preferred_element_type=jnp.float32)
    out_ref[...] = acc


def kernel(scores, bbox_deltas, im_info):
    raw = jnp.transpose(scores[:, A:, :, :], (0, 2, 3, 1)).reshape(NB, 128)
    d = jnp.transpose(bbox_deltas, (0, 2, 3, 1)).reshape(-1, 4)
    dx = d[:, 0].reshape(NB, 128)
    dy = d[:, 1].reshape(NB, 128)
    dw = d[:, 2].reshape(NB, 128)
    dh = d[:, 3].reshape(NB, 128)
    anc = jnp.asarray(_ALL_ANCHORS)
    ax1 = anc[:, 0].reshape(NB, 128)
    ay1 = anc[:, 1].reshape(NB, 128)
    ax2 = anc[:, 2].reshape(NB, 128)
    ay2 = anc[:, 3].reshape(NB, 128)
    lims = jnp.concatenate([
        im_info[0:1, 1:2] - 1.0,
        im_info[0:1, 0:1] - 1.0,
        MIN_SIZE * im_info[0:1, 2:3],
        jnp.zeros((1, 125), jnp.float32)], axis=1)

    rows, rank = pl.pallas_call(
        _k1_body,
        out_shape=[jax.ShapeDtypeStruct((N, 128), jnp.float32),
                   jax.ShapeDtypeStruct((NB, 128), jnp.int32)],
        scratch_shapes=[pltpu.VMEM((NB, 128), jnp.float32)] * 6,
    )(ax1, ay1, ax2, ay2, dx, dy, dw, dh, raw, lims)

    sorted_perm = _sc_scatter(rows, rank.reshape(N))
    sorted_rows = sorted_perm[:NS, :]

    outp = pl.pallas_call(
        _k3_body,
        out_shape=jax.ShapeDtypeStruct((NOUT, 16), jnp.float32),
        scratch_shapes=[pltpu.VMEM((16, NS), jnp.float32),
                        pltpu.VMEM((128, 128), jnp.float32)],
    )(sorted_rows)

    boxes = outp[:POST_NMS_TOPN, 0:4]
    return jnp.concatenate([jnp.zeros((POST_NMS_TOPN, 1), jnp.float32), boxes],
                           axis=1)
